# Initial kernel scaffold; baseline (speedup 1.0000x reference)
import jax
import jax.numpy as jnp
from jax.experimental import pallas as pl


def kernel(sentence_batch, emb_table, fc_w, fc_b):
    raise NotImplementedError("write your pallas kernel here")


# speedup vs baseline: 4.1746x; 4.1746x over previous
import functools

import jax
import jax.numpy as jnp
from jax import lax
from jax.experimental import pallas as pl
from jax.experimental.pallas import tpu as pltpu
from jax.experimental.pallas import tpu_sc as plsc

B = 16384
L = 20
E = 128
NOUT = 1000
NPAD = 1024

NC = 2
NS = 16
NW = NC * NS
LANES = 16

BPW = B // NW
TPW = BPW * L
SPG = 4
RPG = SPG * L
NG = BPW // SPG

_MESH = plsc.VectorSubcoreMesh(
    core_axis_name="c", subcore_axis_name="s", num_cores=NC, num_subcores=NS)


@functools.partial(
    pl.kernel,
    out_type=jax.ShapeDtypeStruct((B, E), jnp.float32),
    mesh=_MESH,
    scratch_types=[
        pltpu.VMEM((TPW,), jnp.int32),
        pltpu.VMEM((RPG, E), jnp.float32),
        pltpu.VMEM((SPG, E), jnp.float32),
        pltpu.SemaphoreType.DMA,
    ],
)
def _pool(idx_hbm, table_hbm, out_hbm, idx_v, rows_v, pooled_v, sem):
    wid = lax.axis_index("s") * NC + lax.axis_index("c")
    base_tok = wid * TPW
    base_row = wid * BPW
    pltpu.sync_copy(idx_hbm.at[pl.ds(base_tok, TPW)], idx_v)

    def grp(g, carry):
        pltpu.async_copy(
            table_hbm.at[idx_v.at[pl.ds(g * RPG, RPG)]], rows_v, sem).wait()
        for i in range(SPG):
            for v in range(E // LANES):
                acc = rows_v[i * L, pl.ds(v * LANES, LANES)]
                for t in range(1, L):
                    acc = acc + rows_v[i * L + t, pl.ds(v * LANES, LANES)]
                pooled_v[i, pl.ds(v * LANES, LANES)] = acc
        pltpu.sync_copy(pooled_v, out_hbm.at[pl.ds(base_row + g * SPG, SPG)])
        return carry

    lax.fori_loop(0, NG, grp, 0)


BM = 512


def _mm_body(x_ref, w_ref, b_ref, o_ref):
    o_ref[...] = (
        jnp.dot(x_ref[...], w_ref[...], preferred_element_type=jnp.float32)
        * (1.0 / L)
        + b_ref[...]
    )


def _head(pooled, w_pad, b_pad):
    return pl.pallas_call(
        _mm_body,
        grid=(B // BM,),
        in_specs=[
            pl.BlockSpec((BM, E), lambda i: (i, 0)),
            pl.BlockSpec((E, NPAD), lambda i: (0, 0)),
            pl.BlockSpec((1, NPAD), lambda i: (0, 0)),
        ],
        out_specs=pl.BlockSpec((BM, NPAD), lambda i: (i, 0)),
        out_shape=jax.ShapeDtypeStruct((B, NPAD), jnp.float32),
    )(pooled, w_pad, b_pad)


def kernel(sentence_batch, emb_table, fc_w, fc_b):
    idx_flat = sentence_batch.reshape(-1).astype(jnp.int32)
    pooled = _pool(idx_flat, emb_table)
    w_pad = jnp.pad(fc_w, ((0, 0), (0, NPAD - NOUT)))
    b_pad = jnp.pad(fc_b, (0, NPAD - NOUT)).reshape(1, NPAD)
    out = _head(pooled, w_pad, b_pad)
    return out[:, :NOUT]


# speedup vs baseline: 4.5300x; 1.0851x over previous
import functools

import jax
import jax.numpy as jnp
from jax import lax
from jax.experimental import pallas as pl
from jax.experimental.pallas import tpu as pltpu
from jax.experimental.pallas import tpu_sc as plsc

B = 16384
L = 20
E = 128
NOUT = 1000

NC = 2
NS = 16
NW = NC * NS
LANES = 16

BPW = B // NW
TPW = BPW * L
SPG = 4
RPG = SPG * L
NG = BPW // SPG
NBUF = 2

_MESH = plsc.VectorSubcoreMesh(
    core_axis_name="c", subcore_axis_name="s", num_cores=NC, num_subcores=NS)


@functools.partial(
    pl.kernel,
    out_type=jax.ShapeDtypeStruct((B, E), jnp.float32),
    mesh=_MESH,
    scratch_types=[
        pltpu.VMEM((TPW,), jnp.int32),
        [pltpu.VMEM((RPG, E), jnp.float32)] * NBUF,
        [pltpu.VMEM((SPG, E), jnp.float32)] * NBUF,
        [pltpu.SemaphoreType.DMA] * NBUF,
        [pltpu.SemaphoreType.DMA] * NBUF,
    ],
)
def _pool(idx_hbm, table_hbm, out_hbm, idx_v, rows_v, pooled_v, gsem, fsem):
    wid = lax.axis_index("s") * NC + lax.axis_index("c")
    base_tok = wid * TPW
    base_row = wid * BPW
    pltpu.sync_copy(idx_hbm.at[pl.ds(base_tok, TPW)], idx_v)

    def start_gather(g, b):
        pltpu.async_copy(
            table_hbm.at[idx_v.at[pl.ds(g * RPG, RPG)]], rows_v[b], gsem[b])

    def wait_gather(b):
        pltpu.make_async_copy(
            table_hbm.at[idx_v.at[pl.ds(0, RPG)]], rows_v[b], gsem[b]).wait()

    def wait_flush(b):
        pltpu.make_async_copy(
            pooled_v[b], out_hbm.at[pl.ds(base_row, SPG)], fsem[b]).wait()

    for b in range(NBUF):
        start_gather(b, b)

    def step(gg, carry):
        for b in range(NBUF):
            g = gg * NBUF + b
            wait_gather(b)
            @pl.when(g >= NBUF)
            def _():
                wait_flush(b)
            for i in range(SPG):
                for v in range(E // LANES):
                    acc = rows_v[b][i * L, pl.ds(v * LANES, LANES)]
                    for t in range(1, L):
                        acc = acc + rows_v[b][i * L + t, pl.ds(v * LANES, LANES)]
                    pooled_v[b][i, pl.ds(v * LANES, LANES)] = acc
            @pl.when(g + NBUF < NG)
            def _():
                start_gather(g + NBUF, b)
            pltpu.async_copy(
                pooled_v[b], out_hbm.at[pl.ds(base_row + g * SPG, SPG)], fsem[b])
        return carry

    lax.fori_loop(0, NG // NBUF, step, 0)
    for b in range(NBUF):
        wait_flush(b)


BM = 512


def _mm_body(x_ref, w_ref, b_ref, o_ref):
    o_ref[...] = (
        jnp.dot(x_ref[...], w_ref[...], preferred_element_type=jnp.float32)
        * (1.0 / L)
        + b_ref[...]
    )


def _head(pooled, fc_w, fc_b):
    return pl.pallas_call(
        _mm_body,
        grid=(B // BM,),
        in_specs=[
            pl.BlockSpec((BM, E), lambda i: (i, 0)),
            pl.BlockSpec((E, NOUT), lambda i: (0, 0)),
            pl.BlockSpec((1, NOUT), lambda i: (0, 0)),
        ],
        out_specs=pl.BlockSpec((BM, NOUT), lambda i: (i, 0)),
        out_shape=jax.ShapeDtypeStruct((B, NOUT), jnp.float32),
    )(pooled, fc_w, fc_b)


def kernel(sentence_batch, emb_table, fc_w, fc_b):
    idx_flat = sentence_batch.reshape(-1).astype(jnp.int32)
    pooled = _pool(idx_flat, emb_table)
    return _head(pooled, fc_w, fc_b.reshape(1, NOUT))
